# baseline (speedup 1.0000x reference)
import functools

import jax
import jax.numpy as jnp
from jax import lax
from jax.experimental import pallas as pl
from jax.experimental.pallas import tpu as pltpu
from jax.experimental.pallas import tpu_sc as plsc

N = 10000
E = 160000
D = 256
_RB = 2000

_NC, _NS = 2, 16
_NW = _NC * _NS
_RW = 320
_NPAD = _NW * _RW
_DP = D // 2
_CE = 4000
_NCHUNK = E // _CE
_VPC = _CE // 16
_G = 256
_MCAP = _CE + 256
_PK = 512


def _pool_body(x_ref, w_ref, b_ref, o_ref):
    acc = jax.lax.dot_general(
        x_ref[...], w_ref[...], (((1,), (1,)), ((), ())),
        preferred_element_type=jnp.float32)
    o_ref[...] = jnp.maximum(acc + b_ref[...], 0.0).astype(jnp.bfloat16)


def _pool_matmul(x, W, b):
    return pl.pallas_call(
        _pool_body,
        grid=(N // _RB,),
        in_specs=[
            pl.BlockSpec((_RB, D), lambda i: (i, 0)),
            pl.BlockSpec((D, D), lambda i: (0, 0)),
            pl.BlockSpec((1, D), lambda i: (0, 0)),
        ],
        out_specs=pl.BlockSpec((_RB, D), lambda i: (i, 0)),
        out_shape=jax.ShapeDtypeStruct((N, D), jnp.bfloat16),
    )(x, W, b.reshape(1, D))


def _out_body(x_ref, ws_ref, mx_ref, mn_ref, we_ref, b_ref, dw_ref, o_ref,
              *, relu):
    acc = jax.lax.dot_general(
        x_ref[...], ws_ref[...], (((1,), (1,)), ((), ())),
        preferred_element_type=jnp.float32)
    mx = mx_ref[...].astype(jnp.float32)
    mn = mn_ref[...].astype(jnp.float32)
    ne = mx < 0.0
    mx = jnp.where(ne, 0.0, mx)
    mn = jnp.where(ne, 0.0, mn)
    eve = jnp.maximum(dw_ref[0, 0] * mx + dw_ref[0, 1] * mn + dw_ref[0, 2],
                      0.0).astype(jnp.bfloat16)
    acc = acc + jax.lax.dot_general(
        eve, we_ref[...], (((1,), (1,)), ((), ())),
        preferred_element_type=jnp.float32)
    acc = acc + b_ref[...]
    if relu:
        acc = jnp.maximum(acc, 0.0)
    o_ref[...] = acc


def _out_matmul(x, Wself, xmax, xmin, Weve, b, dww, dwb, relu):
    dw = jnp.concatenate([dww, dwb]).reshape(1, 3)
    return pl.pallas_call(
        functools.partial(_out_body, relu=relu),
        grid=(N // _RB,),
        in_specs=[
            pl.BlockSpec((_RB, D), lambda i: (i, 0)),
            pl.BlockSpec((D, D), lambda i: (0, 0)),
            pl.BlockSpec((_RB, D), lambda i: (i, 0)),
            pl.BlockSpec((_RB, D), lambda i: (i, 0)),
            pl.BlockSpec((D, D), lambda i: (0, 0)),
            pl.BlockSpec((1, D), lambda i: (0, 0)),
            pl.BlockSpec((1, 3), lambda i: (0, 0), memory_space=pltpu.SMEM),
        ],
        out_specs=pl.BlockSpec((_RB, D), lambda i: (i, 0)),
        out_shape=jax.ShapeDtypeStruct((N, D), jnp.float32),
    )(x, Wself, xmax, xmin, Weve.astype(jnp.bfloat16), b.reshape(1, D), dw)


_INIT_MAX = -1082081408
_INIT_MIN = 2138603384


def _bits(x):
    return plsc.bitcast(x, jnp.bfloat16)


def _sc_minmax_body(h_hbm, src_hbm, dst_hbm, mx_hbm, mn_hbm,
                    amax, amin, dstb, srcb, mlist, idxb, rows, sem):
    wid = lax.axis_index("s") * _NC + lax.axis_index("c")
    lo = wid * _RW

    cmax = jnp.full((16,), _INIT_MAX, jnp.int32)
    cmin = jnp.full((16,), _INIT_MIN, jnp.int32)

    def _inita(i, _):
        for k in range(_DP // 16):
            amax[i, pl.ds(k * 16, 16)] = cmax
            amin[i, pl.ds(k * 16, 16)] = cmin
        return 0
    lax.fori_loop(0, _RW + 1, _inita, 0)

    trash = jax.lax.iota(jnp.int32, 16) + (_MCAP - 16)
    trashval = jnp.full((16,), _RW, jnp.int32)
    lov = jnp.broadcast_to(lo, (16,)).astype(jnp.int32)
    hiv = lov + _RW

    def _chunk(c, _):
        pltpu.sync_copy(dst_hbm.at[pl.ds(c * _CE, _CE)], dstb)
        pltpu.sync_copy(src_hbm.at[pl.ds(c * _CE, _CE)], srcb)

        def _scan(v, cnt):
            dvec = dstb[pl.ds(v * 16, 16)]
            svec = srcb[pl.ds(v * 16, 16)]
            m = (dvec >= lov) & (dvec < hiv)
            mi = m.astype(jnp.int32)
            cs = plsc.cumsum(mi)
            cntv = jnp.broadcast_to(cnt, (16,)).astype(jnp.int32)
            pos = jnp.where(m, cntv + cs - mi, trash)
            plsc.store_scatter(mlist, [pos], svec * _PK + (dvec - lov))
            return cnt + cs[15]

        cnt = lax.fori_loop(0, _VPC, _scan, jnp.int32(0))
        for t in range(_G // 16):
            mlist[pl.ds(cnt + t * 16, 16)] = trashval
        nb = (cnt + _G - 1) // _G
        _ABL = 2

        def _batch(b, _):
            for t in range(_G // 16):
                pk = mlist[pl.ds(b * _G + t * 16, 16)]
                idxb[pl.ds(t * 16, 16)] = pk // _PK
            pltpu.async_copy(h_hbm.at[idxb], rows, sem).wait()
            if _ABL == 2:
                return 0

            def _edge(j, _):
                pk = mlist[pl.ds(b * _G + j, 16)][0]
                dl = pk % _PK
                for k in range(_DP // 16):
                    rv = _bits(rows[j, pl.ds(k * 16, 16)])
                    amax[dl, pl.ds(k * 16, 16)] = plsc.bitcast(
                        jnp.maximum(_bits(amax[dl, pl.ds(k * 16, 16)]), rv),
                        jnp.int32)
                    amin[dl, pl.ds(k * 16, 16)] = plsc.bitcast(
                        jnp.minimum(_bits(amin[dl, pl.ds(k * 16, 16)]), rv),
                        jnp.int32)
                return 0
            lax.fori_loop(0, _G, _edge, 0)
            return 0

        lax.fori_loop(0, nb, _batch, 0)
        return 0

    lax.fori_loop(0, _NCHUNK, _chunk, 0)

    pltpu.sync_copy(amax.at[pl.ds(0, _RW)], mx_hbm.at[pl.ds(lo, _RW)])
    pltpu.sync_copy(amin.at[pl.ds(0, _RW)], mn_hbm.at[pl.ds(lo, _RW)])


def _sc_minmax(hp, src, dst):
    mesh = plsc.VectorSubcoreMesh(core_axis_name="c", subcore_axis_name="s",
                                  num_cores=_NC, num_subcores=_NS)
    run = pl.kernel(
        _sc_minmax_body,
        out_type=(jax.ShapeDtypeStruct((_NPAD, _DP), jnp.int32),
                  jax.ShapeDtypeStruct((_NPAD, _DP), jnp.int32)),
        mesh=mesh,
        scratch_types=[
            pltpu.VMEM((_RW + 1, _DP), jnp.int32),
            pltpu.VMEM((_RW + 1, _DP), jnp.int32),
            pltpu.VMEM((_CE,), jnp.int32),
            pltpu.VMEM((_CE,), jnp.int32),
            pltpu.VMEM((_MCAP,), jnp.int32),
            pltpu.VMEM((_G,), jnp.int32),
            pltpu.VMEM((_G, _DP), jnp.int32),
            pltpu.SemaphoreType.DMA,
        ],
        compiler_params=pltpu.CompilerParams(needs_layout_passes=False),
    )
    return run(hp, src, dst)


def _unpack(a):
    return jax.lax.bitcast_convert_type(
        a, jnp.bfloat16).reshape(_NPAD, D)[:N]


def _layer(x, src, dst, Wpool, bpool, dww, dwb, Weve, Wself, bias, relu):
    h = _pool_matmul(x, Wpool, bpool)
    hp = jax.lax.bitcast_convert_type(h.reshape(N, _DP, 2), jnp.int32)
    mxp, mnp = _sc_minmax(hp, src, dst)
    return _out_matmul(x, Wself, _unpack(mxp), _unpack(mnp), Weve, bias,
                       dww, dwb, relu)


def kernel(x, edge_index, c1_Wpool, c1_bpool, c1_dww, c1_dwb, c1_Weve, c1_Wself, c1_bias, c2_Wpool, c2_bpool, c2_dww, c2_dwb, c2_Weve, c2_Wself, c2_bias):
    src = edge_index[0]
    dst = edge_index[1]
    h = _layer(x, src, dst, c1_Wpool, c1_bpool, c1_dww, c1_dwb, c1_Weve,
               c1_Wself, c1_bias, relu=True)
    return _layer(h, src, dst, c2_Wpool, c2_bpool, c2_dww, c2_dwb, c2_Weve,
                  c2_Wself, c2_bias, relu=False)


# speedup vs baseline: 1.9913x; 1.9913x over previous
import functools

import jax
import jax.numpy as jnp
from jax import lax
from jax.experimental import pallas as pl
from jax.experimental.pallas import tpu as pltpu
from jax.experimental.pallas import tpu_sc as plsc

N = 10000
E = 160000
D = 256
_RB = 2000

_NC, _NS = 2, 16
_NW = _NC * _NS
_RW = 320
_NPAD = _NW * _RW
_DP = D // 2
_CE = 4000
_NCHUNK = E // _CE
_VPC = _CE // 16
_G = 128
_MCAP = _CE + 256
_PK = 512


def _pool_body(x_ref, w_ref, b_ref, o_ref):
    acc = jax.lax.dot_general(
        x_ref[...], w_ref[...], (((1,), (1,)), ((), ())),
        preferred_element_type=jnp.float32)
    o_ref[...] = jnp.maximum(acc + b_ref[...], 0.0).astype(jnp.bfloat16)


def _pool_matmul(x, W, b):
    return pl.pallas_call(
        _pool_body,
        grid=(N // _RB,),
        in_specs=[
            pl.BlockSpec((_RB, D), lambda i: (i, 0)),
            pl.BlockSpec((D, D), lambda i: (0, 0)),
            pl.BlockSpec((1, D), lambda i: (0, 0)),
        ],
        out_specs=pl.BlockSpec((_RB, D), lambda i: (i, 0)),
        out_shape=jax.ShapeDtypeStruct((N, D), jnp.bfloat16),
    )(x, W, b.reshape(1, D))


def _out_body(x_ref, ws_ref, mx_ref, mn_ref, we_ref, b_ref, dw_ref, o_ref,
              *, relu):
    acc = jax.lax.dot_general(
        x_ref[...], ws_ref[...], (((1,), (1,)), ((), ())),
        preferred_element_type=jnp.float32)
    mx = mx_ref[...].astype(jnp.float32)
    mn = mn_ref[...].astype(jnp.float32)
    ne = mx < 0.0
    mx = jnp.where(ne, 0.0, mx)
    mn = jnp.where(ne, 0.0, mn)
    eve = jnp.maximum(dw_ref[0, 0] * mx + dw_ref[0, 1] * mn + dw_ref[0, 2],
                      0.0).astype(jnp.bfloat16)
    acc = acc + jax.lax.dot_general(
        eve, we_ref[...], (((1,), (1,)), ((), ())),
        preferred_element_type=jnp.float32)
    acc = acc + b_ref[...]
    if relu:
        acc = jnp.maximum(acc, 0.0)
    o_ref[...] = acc


def _out_matmul(x, Wself, xmax, xmin, Weve, b, dww, dwb, relu):
    dw = jnp.concatenate([dww, dwb]).reshape(1, 3)
    return pl.pallas_call(
        functools.partial(_out_body, relu=relu),
        grid=(N // _RB,),
        in_specs=[
            pl.BlockSpec((_RB, D), lambda i: (i, 0)),
            pl.BlockSpec((D, D), lambda i: (0, 0)),
            pl.BlockSpec((_RB, D), lambda i: (i, 0)),
            pl.BlockSpec((_RB, D), lambda i: (i, 0)),
            pl.BlockSpec((D, D), lambda i: (0, 0)),
            pl.BlockSpec((1, D), lambda i: (0, 0)),
            pl.BlockSpec((1, 3), lambda i: (0, 0), memory_space=pltpu.SMEM),
        ],
        out_specs=pl.BlockSpec((_RB, D), lambda i: (i, 0)),
        out_shape=jax.ShapeDtypeStruct((N, D), jnp.float32),
    )(x, Wself, xmax, xmin, Weve.astype(jnp.bfloat16), b.reshape(1, D), dw)


_INIT_MAX = -1082081408
_INIT_MIN = 2138603384


def _bits(x):
    return plsc.bitcast(x, jnp.bfloat16)


def _sc_minmax_body(h_hbm, src_hbm, dst_hbm, mx_hbm, mn_hbm,
                    amax, amin, dstb, srcb, mlist, rows, sem):
    wid = lax.axis_index("s") * _NC + lax.axis_index("c")
    lo = wid * _RW

    cmax = jnp.full((16,), _INIT_MAX, jnp.int32)
    cmin = jnp.full((16,), _INIT_MIN, jnp.int32)

    def _inita(i, _):
        for k in range(_DP // 16):
            amax[i, pl.ds(k * 16, 16)] = cmax
            amin[i, pl.ds(k * 16, 16)] = cmin
        return 0
    lax.fori_loop(0, _RW + 1, _inita, 0)

    trash = jax.lax.iota(jnp.int32, 16) + (_MCAP - 16)
    trashval = jnp.full((16,), _RW, jnp.int32)
    lov = jnp.broadcast_to(lo, (16,)).astype(jnp.int32)
    hiv = lov + _RW

    def _chunk(c, _):
        pltpu.sync_copy(dst_hbm.at[pl.ds(c * _CE, _CE)], dstb)
        pltpu.sync_copy(src_hbm.at[pl.ds(c * _CE, _CE)], srcb)

        def _scan(v, cnt):
            dvec = dstb[pl.ds(v * 16, 16)]
            svec = srcb[pl.ds(v * 16, 16)]
            m = (dvec >= lov) & (dvec < hiv)
            mi = m.astype(jnp.int32)
            cs = plsc.cumsum(mi)
            cntv = jnp.broadcast_to(cnt, (16,)).astype(jnp.int32)
            pos = jnp.where(m, cntv + cs - mi, trash)
            plsc.store_scatter(mlist, [pos], svec * _PK + (dvec - lov))
            return cnt + cs[15]

        cnt = lax.fori_loop(0, _VPC, _scan, jnp.int32(0))
        for t in range(_G // 16):
            mlist[pl.ds(cnt + t * 16, 16)] = trashval
        nb = (cnt + _G - 1) // _G

        def _batch(b, _):
            def _fire(j, _):
                pk = mlist[pl.ds(b * _G + j, 16)][0]
                s = pk // _PK
                pltpu.async_copy(h_hbm.at[pl.ds(s * _DP, _DP)],
                                 rows.at[pl.ds(j * _DP, _DP)], sem)
                return 0
            lax.fori_loop(0, _G, _fire, 0)
            pltpu.make_async_copy(h_hbm.at[pl.ds(0, _G * _DP)], rows,
                                  sem).wait()

            def _edge(j, _):
                pk = mlist[pl.ds(b * _G + j, 16)][0]
                dl = pk % _PK
                for k in range(_DP // 16):
                    rv = _bits(rows[pl.ds(j * _DP + k * 16, 16)])
                    amax[dl, pl.ds(k * 16, 16)] = plsc.bitcast(
                        jnp.maximum(_bits(amax[dl, pl.ds(k * 16, 16)]), rv),
                        jnp.int32)
                    amin[dl, pl.ds(k * 16, 16)] = plsc.bitcast(
                        jnp.minimum(_bits(amin[dl, pl.ds(k * 16, 16)]), rv),
                        jnp.int32)
                return 0
            lax.fori_loop(0, _G, _edge, 0)
            return 0

        lax.fori_loop(0, nb, _batch, 0)
        return 0

    lax.fori_loop(0, _NCHUNK, _chunk, 0)

    pltpu.sync_copy(amax.at[pl.ds(0, _RW)], mx_hbm.at[pl.ds(lo, _RW)])
    pltpu.sync_copy(amin.at[pl.ds(0, _RW)], mn_hbm.at[pl.ds(lo, _RW)])


def _sc_minmax(hp, src, dst):
    mesh = plsc.VectorSubcoreMesh(core_axis_name="c", subcore_axis_name="s",
                                  num_cores=_NC, num_subcores=_NS)
    run = pl.kernel(
        _sc_minmax_body,
        out_type=(jax.ShapeDtypeStruct((_NPAD, _DP), jnp.int32),
                  jax.ShapeDtypeStruct((_NPAD, _DP), jnp.int32)),
        mesh=mesh,
        scratch_types=[
            pltpu.VMEM((_RW + 1, _DP), jnp.int32),
            pltpu.VMEM((_RW + 1, _DP), jnp.int32),
            pltpu.VMEM((_CE,), jnp.int32),
            pltpu.VMEM((_CE,), jnp.int32),
            pltpu.VMEM((_MCAP,), jnp.int32),
            pltpu.VMEM((_G * _DP,), jnp.int32),
            pltpu.SemaphoreType.DMA,
        ],
        compiler_params=pltpu.CompilerParams(needs_layout_passes=False),
    )
    return run(hp, src, dst)


def _unpack(a):
    return jax.lax.bitcast_convert_type(
        a, jnp.bfloat16).reshape(_NPAD, D)[:N]


def _layer(x, src, dst, Wpool, bpool, dww, dwb, Weve, Wself, bias, relu):
    h = _pool_matmul(x, Wpool, bpool)
    hp = jax.lax.bitcast_convert_type(h.reshape(N, _DP, 2),
                                      jnp.int32).reshape(N * _DP)
    mxp, mnp = _sc_minmax(hp, src, dst)
    return _out_matmul(x, Wself, _unpack(mxp), _unpack(mnp), Weve, bias,
                       dww, dwb, relu)


def kernel(x, edge_index, c1_Wpool, c1_bpool, c1_dww, c1_dwb, c1_Weve, c1_Wself, c1_bias, c2_Wpool, c2_bpool, c2_dww, c2_dwb, c2_Weve, c2_Wself, c2_bias):
    src = edge_index[0]
    dst = edge_index[1]
    h = _layer(x, src, dst, c1_Wpool, c1_bpool, c1_dww, c1_dwb, c1_Weve,
               c1_Wself, c1_bias, relu=True)
    return _layer(h, src, dst, c2_Wpool, c2_bpool, c2_dww, c2_dwb, c2_Weve,
                  c2_Wself, c2_bias, relu=False)


# speedup vs baseline: 6.9396x; 3.4850x over previous
import functools

import jax
import jax.numpy as jnp
from jax import lax
from jax.experimental import pallas as pl
from jax.experimental.pallas import tpu as pltpu
from jax.experimental.pallas import tpu_sc as plsc

N = 10000
E = 160000
D = 256
_RB = 2000

_NC, _NS = 2, 16
_NW = _NC * _NS
_RW = 320
_NPAD = _NW * _RW
_DP = D // 2
_CE = 4000
_NCHUNK = E // _CE
_VPC = _CE // 16
_PK = 512

_HC = 256
_NT = _NPAD // _HC
_SCAP = 320
_BUFW = _HC * _DP
_ODST = 0
_OSRC = _CE
_OML = 2 * _CE
_SLTR = _NT * _SCAP


def _pool_body(x_ref, w_ref, b_ref, o_ref):
    acc = jax.lax.dot_general(
        x_ref[...], w_ref[...], (((1,), (1,)), ((), ())),
        preferred_element_type=jnp.float32)
    o_ref[...] = jnp.maximum(acc + b_ref[...], 0.0).astype(jnp.bfloat16)


def _pool_matmul(x, W, b):
    return pl.pallas_call(
        _pool_body,
        grid=(N // _RB,),
        in_specs=[
            pl.BlockSpec((_RB, D), lambda i: (i, 0)),
            pl.BlockSpec((D, D), lambda i: (0, 0)),
            pl.BlockSpec((1, D), lambda i: (0, 0)),
        ],
        out_specs=pl.BlockSpec((_RB, D), lambda i: (i, 0)),
        out_shape=jax.ShapeDtypeStruct((N, D), jnp.bfloat16),
    )(x, W, b.reshape(1, D))


def _out_body(x_ref, ws_ref, mx_ref, mn_ref, we_ref, b_ref, dw_ref, o_ref,
              *, relu):
    acc = jax.lax.dot_general(
        x_ref[...], ws_ref[...], (((1,), (1,)), ((), ())),
        preferred_element_type=jnp.float32)
    mx = mx_ref[...].astype(jnp.float32)
    mn = mn_ref[...].astype(jnp.float32)
    ne = mx < 0.0
    mx = jnp.where(ne, 0.0, mx)
    mn = jnp.where(ne, 0.0, mn)
    eve = jnp.maximum(dw_ref[0, 0] * mx + dw_ref[0, 1] * mn + dw_ref[0, 2],
                      0.0).astype(jnp.bfloat16)
    acc = acc + jax.lax.dot_general(
        eve, we_ref[...], (((1,), (1,)), ((), ())),
        preferred_element_type=jnp.float32)
    acc = acc + b_ref[...]
    if relu:
        acc = jnp.maximum(acc, 0.0)
    o_ref[...] = acc


def _out_matmul(x, Wself, xmax, xmin, Weve, b, dww, dwb, relu):
    dw = jnp.concatenate([dww, dwb]).reshape(1, 3)
    return pl.pallas_call(
        functools.partial(_out_body, relu=relu),
        grid=(N // _RB,),
        in_specs=[
            pl.BlockSpec((_RB, D), lambda i: (i, 0)),
            pl.BlockSpec((D, D), lambda i: (0, 0)),
            pl.BlockSpec((_RB, D), lambda i: (i, 0)),
            pl.BlockSpec((_RB, D), lambda i: (i, 0)),
            pl.BlockSpec((D, D), lambda i: (0, 0)),
            pl.BlockSpec((1, D), lambda i: (0, 0)),
            pl.BlockSpec((1, 3), lambda i: (0, 0), memory_space=pltpu.SMEM),
        ],
        out_specs=pl.BlockSpec((_RB, D), lambda i: (i, 0)),
        out_shape=jax.ShapeDtypeStruct((N, D), jnp.float32),
    )(x, Wself, xmax, xmin, Weve.astype(jnp.bfloat16), b.reshape(1, D), dw)


_INIT_MAX = -1082081408
_INIT_MIN = 2138603384


def _bits(x):
    return plsc.bitcast(x, jnp.bfloat16)


def _sc_minmax_body(h_hbm, src_hbm, dst_hbm, mx_hbm, mn_hbm,
                    amax, amin, buf, subl, slowrow, counts, sem):
    wid = lax.axis_index("s") * _NC + lax.axis_index("c")
    lo = wid * _RW

    cmax = jnp.full((16,), _INIT_MAX, jnp.int32)
    cmin = jnp.full((16,), _INIT_MIN, jnp.int32)

    def _inita(i, _):
        for k in range(_DP // 16):
            amax[i, pl.ds(k * 16, 16)] = cmax
            amin[i, pl.ds(k * 16, 16)] = cmin
        return 0
    lax.fori_loop(0, _RW + 1, _inita, 0)

    def _initc(t, _):
        counts[t] = 0
        return 0
    lax.fori_loop(0, _NT, _initc, 0)

    lane = jax.lax.iota(jnp.int32, 16)
    trash = lane + (_OML + _CE)
    sltrash = lane + _SLTR
    lov = jnp.broadcast_to(lo, (16,)).astype(jnp.int32)
    hiv = lov + _RW

    def _acc_row(dl, load_row):
        for k in range(_DP // 16):
            rv = _bits(load_row(k))
            amax[dl, pl.ds(k * 16, 16)] = plsc.bitcast(
                jnp.maximum(_bits(amax[dl, pl.ds(k * 16, 16)]), rv),
                jnp.int32)
            amin[dl, pl.ds(k * 16, 16)] = plsc.bitcast(
                jnp.minimum(_bits(amin[dl, pl.ds(k * 16, 16)]), rv),
                jnp.int32)

    def _chunk(c, _):
        pltpu.sync_copy(dst_hbm.at[pl.ds(c * _CE, _CE)],
                        buf.at[pl.ds(_ODST, _CE)])
        pltpu.sync_copy(src_hbm.at[pl.ds(c * _CE, _CE)],
                        buf.at[pl.ds(_OSRC, _CE)])

        def _scan(v, cnt):
            dvec = buf[pl.ds(_ODST + v * 16, 16)]
            svec = buf[pl.ds(_OSRC + v * 16, 16)]
            m = (dvec >= lov) & (dvec < hiv)
            mi = m.astype(jnp.int32)
            cs = plsc.cumsum(mi)
            cntv = jnp.broadcast_to(cnt, (16,)).astype(jnp.int32)
            pos = jnp.where(m, cntv + (_OML + cs - mi), trash)
            plsc.store_scatter(buf, [pos], svec * _PK + (dvec - lov))
            return cnt + cs[15]

        cnt = lax.fori_loop(0, _VPC, _scan, jnp.int32(0))

        def _distrib(j, _):
            pk = buf[pl.ds(_OML + j, 16)][0]
            t = pk // (_PK * _HC)
            p = counts[t]
            counts[t] = p + 1

            @pl.when(p < _SCAP)
            def _append():
                posv = jnp.where(lane < 1,
                                 jnp.broadcast_to(t * _SCAP + p, (16,)),
                                 sltrash)
                plsc.store_scatter(subl, [posv],
                                   jnp.broadcast_to(pk, (16,)))

            @pl.when(p >= _SCAP)
            def _slow():
                pltpu.async_copy(
                    h_hbm.at[pl.ds((pk // _PK) * _DP, _DP)], slowrow,
                    sem).wait()
                _acc_row(pk % _PK, lambda k: slowrow[pl.ds(k * 16, 16)])
            return 0

        lax.fori_loop(0, cnt, _distrib, 0)
        return 0

    lax.fori_loop(0, _NCHUNK, _chunk, 0)

    def _tile(t, _):
        pltpu.sync_copy(h_hbm.at[pl.ds(t * _BUFW, _BUFW)], buf)
        nj = jnp.minimum(counts[t], _SCAP)

        def _edge(j, _):
            pk = subl[pl.ds(t * _SCAP + j, 16)][0]
            roff = (pk // _PK - t * _HC) * _DP
            _acc_row(pk % _PK, lambda k: buf[pl.ds(roff + k * 16, 16)])
            return 0
        lax.fori_loop(0, nj, _edge, 0)
        return 0

    lax.fori_loop(0, _NT, _tile, 0)

    pltpu.sync_copy(amax.at[pl.ds(0, _RW)], mx_hbm.at[pl.ds(lo, _RW)])
    pltpu.sync_copy(amin.at[pl.ds(0, _RW)], mn_hbm.at[pl.ds(lo, _RW)])


def _sc_minmax(hp, src, dst):
    mesh = plsc.VectorSubcoreMesh(core_axis_name="c", subcore_axis_name="s",
                                  num_cores=_NC, num_subcores=_NS)
    run = pl.kernel(
        _sc_minmax_body,
        out_type=(jax.ShapeDtypeStruct((_NPAD, _DP), jnp.int32),
                  jax.ShapeDtypeStruct((_NPAD, _DP), jnp.int32)),
        mesh=mesh,
        scratch_types=[
            pltpu.VMEM((_RW + 1, _DP), jnp.int32),
            pltpu.VMEM((_RW + 1, _DP), jnp.int32),
            pltpu.VMEM((_BUFW,), jnp.int32),
            pltpu.VMEM((_SLTR + 16,), jnp.int32),
            pltpu.VMEM((_DP,), jnp.int32),
            pltpu.SMEM((_NT,), jnp.int32),
            pltpu.SemaphoreType.DMA,
        ],
        compiler_params=pltpu.CompilerParams(needs_layout_passes=False),
    )
    return run(hp, src, dst)


def _unpack(a):
    return jax.lax.bitcast_convert_type(
        a, jnp.bfloat16).reshape(_NPAD, D)[:N]


def _layer(x, src, dst, Wpool, bpool, dww, dwb, Weve, Wself, bias, relu):
    h = _pool_matmul(x, Wpool, bpool)
    hp = jax.lax.bitcast_convert_type(h.reshape(N, _DP, 2),
                                      jnp.int32).reshape(N * _DP)
    hp = jnp.pad(hp, (0, (_NPAD - N) * _DP))
    mxp, mnp = _sc_minmax(hp, src, dst)
    return _out_matmul(x, Wself, _unpack(mxp), _unpack(mnp), Weve, bias,
                       dww, dwb, relu)


def kernel(x, edge_index, c1_Wpool, c1_bpool, c1_dww, c1_dwb, c1_Weve, c1_Wself, c1_bias, c2_Wpool, c2_bpool, c2_dww, c2_dwb, c2_Weve, c2_Wself, c2_bias):
    src = edge_index[0]
    dst = edge_index[1]
    h = _layer(x, src, dst, c1_Wpool, c1_bpool, c1_dww, c1_dwb, c1_Weve,
               c1_Wself, c1_bias, relu=True)
    return _layer(h, src, dst, c2_Wpool, c2_bpool, c2_dww, c2_dwb, c2_Weve,
                  c2_Wself, c2_bias, relu=False)


# speedup vs baseline: 7.5389x; 1.0863x over previous
import functools

import jax
import jax.numpy as jnp
from jax import lax
from jax.experimental import pallas as pl
from jax.experimental.pallas import tpu as pltpu
from jax.experimental.pallas import tpu_sc as plsc

N = 10000
E = 160000
D = 256
_RB = 2000

_NC, _NS = 2, 16
_NW = _NC * _NS
_RW = 320
_NPAD = _NW * _RW
_DP = D // 2
_CE = 6400
_NCHUNK = E // _CE
_UNR = 4
_VPC = _CE // (16 * _UNR)
_PK = 512

_HC = 256
_NT = _NPAD // _HC
_SCAP = 320
_BUFW = _HC * _DP
_ODST = 0
_OSRC = _CE
_OML = 2 * _CE
_SLTR = _NT * _SCAP


def _pool_body(x_ref, w_ref, b_ref, o_ref):
    acc = jax.lax.dot_general(
        x_ref[...], w_ref[...], (((1,), (1,)), ((), ())),
        preferred_element_type=jnp.float32)
    o_ref[...] = jnp.maximum(acc + b_ref[...], 0.0).astype(jnp.bfloat16)


def _pool_matmul(x, W, b):
    return pl.pallas_call(
        _pool_body,
        grid=(N // _RB,),
        in_specs=[
            pl.BlockSpec((_RB, D), lambda i: (i, 0)),
            pl.BlockSpec((D, D), lambda i: (0, 0)),
            pl.BlockSpec((1, D), lambda i: (0, 0)),
        ],
        out_specs=pl.BlockSpec((_RB, D), lambda i: (i, 0)),
        out_shape=jax.ShapeDtypeStruct((N, D), jnp.bfloat16),
    )(x, W, b.reshape(1, D))


def _out_body(x_ref, ws_ref, mx_ref, mn_ref, we_ref, b_ref, dw_ref, o_ref,
              *, relu):
    acc = jax.lax.dot_general(
        x_ref[...], ws_ref[...], (((1,), (1,)), ((), ())),
        preferred_element_type=jnp.float32)
    mx = mx_ref[...].astype(jnp.float32)
    mn = mn_ref[...].astype(jnp.float32)
    ne = mx < 0.0
    mx = jnp.where(ne, 0.0, mx)
    mn = jnp.where(ne, 0.0, mn)
    eve = jnp.maximum(dw_ref[0, 0] * mx + dw_ref[0, 1] * mn + dw_ref[0, 2],
                      0.0).astype(jnp.bfloat16)
    acc = acc + jax.lax.dot_general(
        eve, we_ref[...], (((1,), (1,)), ((), ())),
        preferred_element_type=jnp.float32)
    acc = acc + b_ref[...]
    if relu:
        acc = jnp.maximum(acc, 0.0)
    o_ref[...] = acc


def _out_matmul(x, Wself, xmax, xmin, Weve, b, dww, dwb, relu):
    dw = jnp.concatenate([dww, dwb]).reshape(1, 3)
    return pl.pallas_call(
        functools.partial(_out_body, relu=relu),
        grid=(N // _RB,),
        in_specs=[
            pl.BlockSpec((_RB, D), lambda i: (i, 0)),
            pl.BlockSpec((D, D), lambda i: (0, 0)),
            pl.BlockSpec((_RB, D), lambda i: (i, 0)),
            pl.BlockSpec((_RB, D), lambda i: (i, 0)),
            pl.BlockSpec((D, D), lambda i: (0, 0)),
            pl.BlockSpec((1, D), lambda i: (0, 0)),
            pl.BlockSpec((1, 3), lambda i: (0, 0), memory_space=pltpu.SMEM),
        ],
        out_specs=pl.BlockSpec((_RB, D), lambda i: (i, 0)),
        out_shape=jax.ShapeDtypeStruct((N, D), jnp.float32),
    )(x, Wself, xmax, xmin, Weve.astype(jnp.bfloat16), b.reshape(1, D), dw)


_INIT_MAX = -1082081408
_INIT_MIN = 2138603384


def _bits(x):
    return plsc.bitcast(x, jnp.bfloat16)


def _sc_minmax_body(h_hbm, src_hbm, dst_hbm, mx_hbm, mn_hbm,
                    amax, amin, buf, subl, slowrow, counts, sem):
    wid = lax.axis_index("s") * _NC + lax.axis_index("c")
    lo = wid * _RW

    cmax = jnp.full((16,), _INIT_MAX, jnp.int32)
    cmin = jnp.full((16,), _INIT_MIN, jnp.int32)

    def _inita(i, _):
        for k in range(_DP // 16):
            amax[i, pl.ds(k * 16, 16)] = cmax
            amin[i, pl.ds(k * 16, 16)] = cmin
        return 0
    lax.fori_loop(0, _RW + 1, _inita, 0)

    def _initc(t, _):
        counts[t] = 0
        return 0
    lax.fori_loop(0, _NT, _initc, 0)

    lane = jax.lax.iota(jnp.int32, 16)
    trash = lane + (_OML + _CE)
    sltrash = lane + _SLTR
    lov = jnp.broadcast_to(lo, (16,)).astype(jnp.int32)
    hiv = lov + _RW

    def _acc_row(dl, load_row):
        for k in range(_DP // 16):
            rv = _bits(load_row(k))
            amax[dl, pl.ds(k * 16, 16)] = plsc.bitcast(
                jnp.maximum(_bits(amax[dl, pl.ds(k * 16, 16)]), rv),
                jnp.int32)
            amin[dl, pl.ds(k * 16, 16)] = plsc.bitcast(
                jnp.minimum(_bits(amin[dl, pl.ds(k * 16, 16)]), rv),
                jnp.int32)

    def _chunk(c, _):
        pltpu.sync_copy(dst_hbm.at[pl.ds(c * _CE, _CE)],
                        buf.at[pl.ds(_ODST, _CE)])
        pltpu.sync_copy(src_hbm.at[pl.ds(c * _CE, _CE)],
                        buf.at[pl.ds(_OSRC, _CE)])

        def _scan(v, cnt):
            parts = []
            for u in range(_UNR):
                o = v * 16 * _UNR + u * 16
                dvec = buf[pl.ds(_ODST + o, 16)]
                svec = buf[pl.ds(_OSRC + o, 16)]
                m = (dvec >= lov) & (dvec < hiv)
                mi = m.astype(jnp.int32)
                cs = plsc.cumsum(mi)
                parts.append((m, mi, cs, svec * _PK + (dvec - lov)))
            for m, mi, cs, val in parts:
                cntv = jnp.broadcast_to(cnt, (16,)).astype(jnp.int32)
                pos = jnp.where(m, cntv + (_OML + cs - mi), trash)
                plsc.store_scatter(buf, [pos], val)
                cnt = cnt + cs[15]
            return cnt

        cnt = lax.fori_loop(0, _VPC, _scan, jnp.int32(0))

        def _distrib(j, _):
            pk = buf[pl.ds(_OML + j, 16)][0]
            t = pk // (_PK * _HC)
            p = counts[t]
            counts[t] = p + 1

            @pl.when(p < _SCAP)
            def _append():
                posv = jnp.where(lane < 1,
                                 jnp.broadcast_to(t * _SCAP + p, (16,)),
                                 sltrash)
                plsc.store_scatter(subl, [posv],
                                   jnp.broadcast_to(pk, (16,)))

            @pl.when(p >= _SCAP)
            def _slow():
                pltpu.async_copy(
                    h_hbm.at[pl.ds((pk // _PK) * _DP, _DP)], slowrow,
                    sem).wait()
                _acc_row(pk % _PK, lambda k: slowrow[pl.ds(k * 16, 16)])
            return 0

        lax.fori_loop(0, cnt, _distrib, 0)
        return 0

    lax.fori_loop(0, _NCHUNK, _chunk, 0)

    def _tile(t, _):
        pltpu.sync_copy(h_hbm.at[pl.ds(t * _BUFW, _BUFW)], buf)
        nj = jnp.minimum(counts[t], _SCAP)

        def _edge(j, _):
            pk = subl[pl.ds(t * _SCAP + j, 16)][0]
            roff = (pk // _PK - t * _HC) * _DP
            _acc_row(pk % _PK, lambda k: buf[pl.ds(roff + k * 16, 16)])
            return 0
        lax.fori_loop(0, nj, _edge, 0)
        return 0

    lax.fori_loop(0, _NT, _tile, 0)

    pltpu.sync_copy(amax.at[pl.ds(0, _RW)], mx_hbm.at[pl.ds(lo, _RW)])
    pltpu.sync_copy(amin.at[pl.ds(0, _RW)], mn_hbm.at[pl.ds(lo, _RW)])


def _sc_minmax(hp, src, dst):
    mesh = plsc.VectorSubcoreMesh(core_axis_name="c", subcore_axis_name="s",
                                  num_cores=_NC, num_subcores=_NS)
    run = pl.kernel(
        _sc_minmax_body,
        out_type=(jax.ShapeDtypeStruct((_NPAD, _DP), jnp.int32),
                  jax.ShapeDtypeStruct((_NPAD, _DP), jnp.int32)),
        mesh=mesh,
        scratch_types=[
            pltpu.VMEM((_RW + 1, _DP), jnp.int32),
            pltpu.VMEM((_RW + 1, _DP), jnp.int32),
            pltpu.VMEM((_BUFW,), jnp.int32),
            pltpu.VMEM((_SLTR + 16,), jnp.int32),
            pltpu.VMEM((_DP,), jnp.int32),
            pltpu.SMEM((_NT,), jnp.int32),
            pltpu.SemaphoreType.DMA,
        ],
        compiler_params=pltpu.CompilerParams(needs_layout_passes=False),
    )
    return run(hp, src, dst)


def _unpack(a):
    return jax.lax.bitcast_convert_type(
        a, jnp.bfloat16).reshape(_NPAD, D)[:N]


def _layer(x, src, dst, Wpool, bpool, dww, dwb, Weve, Wself, bias, relu):
    h = _pool_matmul(x, Wpool, bpool)
    hp = jax.lax.bitcast_convert_type(h.reshape(N, _DP, 2),
                                      jnp.int32).reshape(N * _DP)
    hp = jnp.pad(hp, (0, (_NPAD - N) * _DP))
    mxp, mnp = _sc_minmax(hp, src, dst)
    return _out_matmul(x, Wself, _unpack(mxp), _unpack(mnp), Weve, bias,
                       dww, dwb, relu)


def kernel(x, edge_index, c1_Wpool, c1_bpool, c1_dww, c1_dwb, c1_Weve, c1_Wself, c1_bias, c2_Wpool, c2_bpool, c2_dww, c2_dwb, c2_Weve, c2_Wself, c2_bias):
    src = edge_index[0]
    dst = edge_index[1]
    h = _layer(x, src, dst, c1_Wpool, c1_bpool, c1_dww, c1_dwb, c1_Weve,
               c1_Wself, c1_bias, relu=True)
    return _layer(h, src, dst, c2_Wpool, c2_bpool, c2_dww, c2_dwb, c2_Weve,
                  c2_Wself, c2_bias, relu=False)


# speedup vs baseline: 8.2568x; 1.0952x over previous
import functools

import jax
import jax.numpy as jnp
from jax import lax
from jax.experimental import pallas as pl
from jax.experimental.pallas import tpu as pltpu
from jax.experimental.pallas import tpu_sc as plsc

N = 10000
E = 160000
D = 256
_RB = 2000

_NC, _NS = 2, 16
_NW = _NC * _NS
_RW = 320
_NPAD = _NW * _RW
_DP = D // 2
_CE = 6400
_NCHUNK = E // _CE
_UNR = 4
_VPC = _CE // (16 * _UNR)
_PK = 512

_HC = 128
_NT = _NPAD // _HC
_SCAP = 160
_HW = _HC * _DP
_BUFW = 2 * _HW
_OML = 4 * _CE
_SLTR = _NT * _SCAP


def _pool_body(x_ref, w_ref, b_ref, o_ref):
    acc = jax.lax.dot_general(
        x_ref[...], w_ref[...], (((1,), (1,)), ((), ())),
        preferred_element_type=jnp.float32)
    o_ref[...] = jnp.maximum(acc + b_ref[...], 0.0).astype(jnp.bfloat16)


def _pool_matmul(x, W, b):
    return pl.pallas_call(
        _pool_body,
        grid=(N // _RB,),
        in_specs=[
            pl.BlockSpec((_RB, D), lambda i: (i, 0)),
            pl.BlockSpec((D, D), lambda i: (0, 0)),
            pl.BlockSpec((1, D), lambda i: (0, 0)),
        ],
        out_specs=pl.BlockSpec((_RB, D), lambda i: (i, 0)),
        out_shape=jax.ShapeDtypeStruct((N, D), jnp.bfloat16),
    )(x, W, b.reshape(1, D))


def _out_body(x_ref, ws_ref, mx_ref, mn_ref, we_ref, b_ref, dw_ref, o_ref,
              *, relu):
    acc = jax.lax.dot_general(
        x_ref[...], ws_ref[...], (((1,), (1,)), ((), ())),
        preferred_element_type=jnp.float32)
    mx = mx_ref[...].astype(jnp.float32)
    mn = mn_ref[...].astype(jnp.float32)
    ne = mx < 0.0
    mx = jnp.where(ne, 0.0, mx)
    mn = jnp.where(ne, 0.0, mn)
    eve = jnp.maximum(dw_ref[0, 0] * mx + dw_ref[0, 1] * mn + dw_ref[0, 2],
                      0.0).astype(jnp.bfloat16)
    acc = acc + jax.lax.dot_general(
        eve, we_ref[...], (((1,), (1,)), ((), ())),
        preferred_element_type=jnp.float32)
    acc = acc + b_ref[...]
    if relu:
        acc = jnp.maximum(acc, 0.0)
    o_ref[...] = acc


def _out_matmul(x, Wself, xmax, xmin, Weve, b, dww, dwb, relu):
    dw = jnp.concatenate([dww, dwb]).reshape(1, 3)
    return pl.pallas_call(
        functools.partial(_out_body, relu=relu),
        grid=(N // _RB,),
        in_specs=[
            pl.BlockSpec((_RB, D), lambda i: (i, 0)),
            pl.BlockSpec((D, D), lambda i: (0, 0)),
            pl.BlockSpec((_RB, D), lambda i: (i, 0)),
            pl.BlockSpec((_RB, D), lambda i: (i, 0)),
            pl.BlockSpec((D, D), lambda i: (0, 0)),
            pl.BlockSpec((1, D), lambda i: (0, 0)),
            pl.BlockSpec((1, 3), lambda i: (0, 0), memory_space=pltpu.SMEM),
        ],
        out_specs=pl.BlockSpec((_RB, D), lambda i: (i, 0)),
        out_shape=jax.ShapeDtypeStruct((N, D), jnp.float32),
    )(x, Wself, xmax, xmin, Weve.astype(jnp.bfloat16), b.reshape(1, D), dw)


_INIT_MAX = -1082081408
_INIT_MIN = 2138603384


def _bits(x):
    return plsc.bitcast(x, jnp.bfloat16)


def _sc_minmax_body(h_hbm, src_hbm, dst_hbm, mx_hbm, mn_hbm,
                    amax, amin, buf, subl, slowrow, counts, sem):
    wid = lax.axis_index("s") * _NC + lax.axis_index("c")
    lo = wid * _RW

    cmax = jnp.full((16,), _INIT_MAX, jnp.int32)
    cmin = jnp.full((16,), _INIT_MIN, jnp.int32)

    def _inita(i, _):
        for k in range(_DP // 16):
            amax[i, pl.ds(k * 16, 16)] = cmax
            amin[i, pl.ds(k * 16, 16)] = cmin
        return 0
    lax.fori_loop(0, _RW + 1, _inita, 0)

    def _initc(t, _):
        counts[t] = 0
        return 0
    lax.fori_loop(0, _NT, _initc, 0)

    lane = jax.lax.iota(jnp.int32, 16)
    trash = lane + (_OML + _CE)
    sltrash = lane + _SLTR
    lov = jnp.broadcast_to(lo, (16,)).astype(jnp.int32)
    hiv = lov + _RW

    def _acc_row(dl, load_row):
        for k in range(_DP // 16):
            rv = _bits(load_row(k))
            amax[dl, pl.ds(k * 16, 16)] = plsc.bitcast(
                jnp.maximum(_bits(amax[dl, pl.ds(k * 16, 16)]), rv),
                jnp.int32)
            amin[dl, pl.ds(k * 16, 16)] = plsc.bitcast(
                jnp.minimum(_bits(amin[dl, pl.ds(k * 16, 16)]), rv),
                jnp.int32)

    def _fetch_chunk(c):
        s = (c % 2) * 2 * _CE
        pltpu.async_copy(dst_hbm.at[pl.ds(c * _CE, _CE)],
                         buf.at[pl.ds(s, _CE)], sem)
        pltpu.async_copy(src_hbm.at[pl.ds(c * _CE, _CE)],
                         buf.at[pl.ds(s + _CE, _CE)], sem)

    _fetch_chunk(0)

    def _chunk(c, _):
        pltpu.make_async_copy(dst_hbm.at[pl.ds(0, 2 * _CE)],
                              buf.at[pl.ds(0, 2 * _CE)], sem).wait()

        @pl.when(c + 1 < _NCHUNK)
        def _():
            _fetch_chunk(c + 1)

        base = (c % 2) * 2 * _CE

        def _scan(v, cnt):
            parts = []
            for u in range(_UNR):
                o = v * 16 * _UNR + u * 16
                dvec = buf[pl.ds(base + o, 16)]
                svec = buf[pl.ds(base + _CE + o, 16)]
                m = (dvec >= lov) & (dvec < hiv)
                mi = m.astype(jnp.int32)
                cs = plsc.cumsum(mi)
                parts.append((m, mi, cs, svec * _PK + (dvec - lov)))
            for m, mi, cs, val in parts:
                cntv = jnp.broadcast_to(cnt, (16,)).astype(jnp.int32)
                pos = jnp.where(m, cntv + (_OML + cs - mi), trash)
                plsc.store_scatter(buf, [pos], val)
                cnt = cnt + cs[15]
            return cnt

        cnt = lax.fori_loop(0, _VPC, _scan, jnp.int32(0))

        def _distrib(j, _):
            pk = buf[pl.ds(_OML + j, 16)][0]
            t = pk // (_PK * _HC)
            p = counts[t]
            counts[t] = p + 1

            @pl.when(p < _SCAP)
            def _append():
                posv = jnp.where(lane < 1,
                                 jnp.broadcast_to(t * _SCAP + p, (16,)),
                                 sltrash)
                plsc.store_scatter(subl, [posv],
                                   jnp.broadcast_to(pk, (16,)))

            @pl.when(p >= _SCAP)
            def _slow():
                pltpu.async_copy(
                    h_hbm.at[pl.ds((pk // _PK) * _DP, _DP)], slowrow,
                    sem).wait()
                _acc_row(pk % _PK, lambda k: slowrow[pl.ds(k * 16, 16)])
            return 0

        lax.fori_loop(0, cnt, _distrib, 0)
        return 0

    lax.fori_loop(0, _NCHUNK, _chunk, 0)

    pltpu.async_copy(h_hbm.at[pl.ds(0, _HW)], buf.at[pl.ds(0, _HW)], sem)

    def _tile(t, _):
        half = (t % 2) * _HW
        pltpu.make_async_copy(h_hbm.at[pl.ds(0, _HW)],
                              buf.at[pl.ds(0, _HW)], sem).wait()

        @pl.when(t + 1 < _NT)
        def _():
            nhalf = ((t + 1) % 2) * _HW
            pltpu.async_copy(h_hbm.at[pl.ds((t + 1) * _HW, _HW)],
                             buf.at[pl.ds(nhalf, _HW)], sem)

        nj = jnp.minimum(counts[t], _SCAP)

        def _edge(j, _):
            pk = subl[pl.ds(t * _SCAP + j, 16)][0]
            roff = (pk // _PK - t * _HC) * _DP + half
            _acc_row(pk % _PK, lambda k: buf[pl.ds(roff + k * 16, 16)])
            return 0
        lax.fori_loop(0, nj, _edge, 0)
        return 0

    lax.fori_loop(0, _NT, _tile, 0)

    pltpu.sync_copy(amax.at[pl.ds(0, _RW)], mx_hbm.at[pl.ds(lo, _RW)])
    pltpu.sync_copy(amin.at[pl.ds(0, _RW)], mn_hbm.at[pl.ds(lo, _RW)])


def _sc_minmax(hp, src, dst):
    mesh = plsc.VectorSubcoreMesh(core_axis_name="c", subcore_axis_name="s",
                                  num_cores=_NC, num_subcores=_NS)
    run = pl.kernel(
        _sc_minmax_body,
        out_type=(jax.ShapeDtypeStruct((_NPAD, _DP), jnp.int32),
                  jax.ShapeDtypeStruct((_NPAD, _DP), jnp.int32)),
        mesh=mesh,
        scratch_types=[
            pltpu.VMEM((_RW + 1, _DP), jnp.int32),
            pltpu.VMEM((_RW + 1, _DP), jnp.int32),
            pltpu.VMEM((_BUFW,), jnp.int32),
            pltpu.VMEM((_SLTR + 16,), jnp.int32),
            pltpu.VMEM((_DP,), jnp.int32),
            pltpu.SMEM((_NT,), jnp.int32),
            pltpu.SemaphoreType.DMA,
        ],
        compiler_params=pltpu.CompilerParams(needs_layout_passes=False),
    )
    return run(hp, src, dst)


def _unpack(a):
    return jax.lax.bitcast_convert_type(
        a, jnp.bfloat16).reshape(_NPAD, D)[:N]


def _layer(x, src, dst, Wpool, bpool, dww, dwb, Weve, Wself, bias, relu):
    h = _pool_matmul(x, Wpool, bpool)
    hp = jax.lax.bitcast_convert_type(h.reshape(N, _DP, 2),
                                      jnp.int32).reshape(N * _DP)
    hp = jnp.pad(hp, (0, (_NPAD - N) * _DP))
    mxp, mnp = _sc_minmax(hp, src, dst)
    return _out_matmul(x, Wself, _unpack(mxp), _unpack(mnp), Weve, bias,
                       dww, dwb, relu)


def kernel(x, edge_index, c1_Wpool, c1_bpool, c1_dww, c1_dwb, c1_Weve, c1_Wself, c1_bias, c2_Wpool, c2_bpool, c2_dww, c2_dwb, c2_Weve, c2_Wself, c2_bias):
    src = edge_index[0]
    dst = edge_index[1]
    h = _layer(x, src, dst, c1_Wpool, c1_bpool, c1_dww, c1_dwb, c1_Weve,
               c1_Wself, c1_bias, relu=True)
    return _layer(h, src, dst, c2_Wpool, c2_bpool, c2_dww, c2_dwb, c2_Weve,
                  c2_Wself, c2_bias, relu=False)


# speedup vs baseline: 8.5023x; 1.0297x over previous
import functools

import jax
import jax.numpy as jnp
from jax import lax
from jax.experimental import pallas as pl
from jax.experimental.pallas import tpu as pltpu
from jax.experimental.pallas import tpu_sc as plsc

N = 10000
E = 160000
D = 256
_RB = 2000

_NC, _NS = 2, 16
_NW = _NC * _NS
_RW = 320
_NPAD = _NW * _RW
_DP = D // 2
_CE = 6400
_NCHUNK = E // _CE
_UNR = 4
_VPC = _CE // (16 * _UNR)
_PK = 512

_HC = 128
_NT = _NPAD // _HC
_SCAP = 160
_HW = _HC * _DP
_BUFW = 2 * _HW
_OML = 4 * _CE
_SLTR = _NT * _SCAP


def _pool_body(x_ref, w_ref, b_ref, o_ref):
    acc = jax.lax.dot_general(
        x_ref[...], w_ref[...], (((1,), (1,)), ((), ())),
        preferred_element_type=jnp.float32)
    o_ref[...] = jnp.maximum(acc + b_ref[...], 0.0).astype(jnp.bfloat16)


def _pool_matmul(x, W, b):
    return pl.pallas_call(
        _pool_body,
        grid=(N // _RB,),
        in_specs=[
            pl.BlockSpec((_RB, D), lambda i: (i, 0)),
            pl.BlockSpec((D, D), lambda i: (0, 0)),
            pl.BlockSpec((1, D), lambda i: (0, 0)),
        ],
        out_specs=pl.BlockSpec((_RB, D), lambda i: (i, 0)),
        out_shape=jax.ShapeDtypeStruct((N, D), jnp.bfloat16),
    )(x, W, b.reshape(1, D))


def _out_body(x_ref, ws_ref, mx_ref, mn_ref, we_ref, b_ref, dw_ref, o_ref,
              *, relu):
    acc = jax.lax.dot_general(
        x_ref[...], ws_ref[...], (((1,), (1,)), ((), ())),
        preferred_element_type=jnp.float32)
    mx = mx_ref[...].astype(jnp.float32)
    mn = mn_ref[...].astype(jnp.float32)
    ne = mx < 0.0
    mx = jnp.where(ne, 0.0, mx)
    mn = jnp.where(ne, 0.0, mn)
    eve = jnp.maximum(dw_ref[0, 0] * mx + dw_ref[0, 1] * mn + dw_ref[0, 2],
                      0.0).astype(jnp.bfloat16)
    acc = acc + jax.lax.dot_general(
        eve, we_ref[...], (((1,), (1,)), ((), ())),
        preferred_element_type=jnp.float32)
    acc = acc + b_ref[...]
    if relu:
        acc = jnp.maximum(acc, 0.0)
    o_ref[...] = acc


def _out_matmul(x, Wself, xmax, xmin, Weve, b, dww, dwb, relu):
    dw = jnp.concatenate([dww, dwb]).reshape(1, 3)
    return pl.pallas_call(
        functools.partial(_out_body, relu=relu),
        grid=(N // _RB,),
        in_specs=[
            pl.BlockSpec((_RB, D), lambda i: (i, 0)),
            pl.BlockSpec((D, D), lambda i: (0, 0)),
            pl.BlockSpec((_RB, D), lambda i: (i, 0)),
            pl.BlockSpec((_RB, D), lambda i: (i, 0)),
            pl.BlockSpec((D, D), lambda i: (0, 0)),
            pl.BlockSpec((1, D), lambda i: (0, 0)),
            pl.BlockSpec((1, 3), lambda i: (0, 0), memory_space=pltpu.SMEM),
        ],
        out_specs=pl.BlockSpec((_RB, D), lambda i: (i, 0)),
        out_shape=jax.ShapeDtypeStruct((N, D), jnp.float32),
    )(x, Wself, xmax, xmin, Weve.astype(jnp.bfloat16), b.reshape(1, D), dw)


_INIT_MAX = -1082081408
_INIT_MIN = 2138603384


def _bits(x):
    return plsc.bitcast(x, jnp.bfloat16)


def _sc_minmax_body(h_hbm, src_hbm, dst_hbm, mx_hbm, mn_hbm,
                    amax, amin, buf, subl, slowrow, counts, sem):
    wid = lax.axis_index("s") * _NC + lax.axis_index("c")
    lo = wid * _RW

    cmax = jnp.full((16,), _INIT_MAX, jnp.int32)
    cmin = jnp.full((16,), _INIT_MIN, jnp.int32)

    def _inita(i, _):
        for k in range(_DP // 16):
            amax[i, pl.ds(k * 16, 16)] = cmax
            amin[i, pl.ds(k * 16, 16)] = cmin
        return 0
    lax.fori_loop(0, _RW + 1, _inita, 0)

    def _initc(t, _):
        counts[t] = 0
        return 0
    lax.fori_loop(0, _NT, _initc, 0)

    lane = jax.lax.iota(jnp.int32, 16)
    trash = lane + (_OML + _CE)
    sltrash = lane + _SLTR
    lov = jnp.broadcast_to(lo, (16,)).astype(jnp.int32)
    hiv = lov + _RW

    def _acc_row(dl, load_row):
        for k in range(_DP // 16):
            rv = _bits(load_row(k))
            amax[dl, pl.ds(k * 16, 16)] = plsc.bitcast(
                jnp.maximum(_bits(amax[dl, pl.ds(k * 16, 16)]), rv),
                jnp.int32)
            amin[dl, pl.ds(k * 16, 16)] = plsc.bitcast(
                jnp.minimum(_bits(amin[dl, pl.ds(k * 16, 16)]), rv),
                jnp.int32)

    def _fetch_chunk(c):
        s = (c % 2) * 2 * _CE
        pltpu.async_copy(dst_hbm.at[pl.ds(c * _CE, _CE)],
                         buf.at[pl.ds(s, _CE)], sem)
        pltpu.async_copy(src_hbm.at[pl.ds(c * _CE, _CE)],
                         buf.at[pl.ds(s + _CE, _CE)], sem)

    _fetch_chunk(0)

    def _chunk(c, _):
        pltpu.make_async_copy(dst_hbm.at[pl.ds(0, 2 * _CE)],
                              buf.at[pl.ds(0, 2 * _CE)], sem).wait()

        @pl.when(c + 1 < _NCHUNK)
        def _():
            _fetch_chunk(c + 1)

        base = (c % 2) * 2 * _CE

        def _scan(v, cnt):
            parts = []
            for u in range(_UNR):
                o = v * 16 * _UNR + u * 16
                dvec = buf[pl.ds(base + o, 16)]
                svec = buf[pl.ds(base + _CE + o, 16)]
                m = (dvec >= lov) & (dvec < hiv)
                mi = m.astype(jnp.int32)
                cs = plsc.cumsum(mi)
                parts.append((m, mi, cs, svec * _PK + (dvec - lov)))
            for m, mi, cs, val in parts:
                cntv = jnp.broadcast_to(cnt, (16,)).astype(jnp.int32)
                pos = jnp.where(m, cntv + (_OML + cs - mi), trash)
                plsc.store_scatter(buf, [pos], val)
                cnt = cnt + cs[15]
            return cnt

        cnt = lax.fori_loop(0, _VPC, _scan, jnp.int32(0))

        def _distrib(j, _):
            pk = buf[pl.ds(_OML + j, 16)][0]
            t = pk // (_PK * _HC)
            p = counts[t]
            counts[t] = p + 1
            slot = jnp.where(p < _SCAP, t * _SCAP + p, _SLTR)
            posv = jnp.where(lane < 1, jnp.broadcast_to(slot, (16,)),
                             sltrash)
            plsc.store_scatter(subl, [posv], jnp.broadcast_to(pk, (16,)))

            @pl.when(p >= _SCAP)
            def _slow():
                pltpu.async_copy(
                    h_hbm.at[pl.ds((pk // _PK) * _DP, _DP)], slowrow,
                    sem).wait()
                _acc_row(pk % _PK, lambda k: slowrow[pl.ds(k * 16, 16)])
            return 0

        lax.fori_loop(0, cnt, _distrib, 0)
        return 0

    lax.fori_loop(0, _NCHUNK, _chunk, 0)

    pltpu.async_copy(h_hbm.at[pl.ds(0, _HW)], buf.at[pl.ds(0, _HW)], sem)

    def _tile(t, _):
        half = (t % 2) * _HW
        pltpu.make_async_copy(h_hbm.at[pl.ds(0, _HW)],
                              buf.at[pl.ds(0, _HW)], sem).wait()

        @pl.when(t + 1 < _NT)
        def _():
            nhalf = ((t + 1) % 2) * _HW
            pltpu.async_copy(h_hbm.at[pl.ds((t + 1) * _HW, _HW)],
                             buf.at[pl.ds(nhalf, _HW)], sem)

        nj = jnp.minimum(counts[t], _SCAP)

        def _edge(j, _):
            pk = subl[pl.ds(t * _SCAP + j, 16)][0]
            roff = (pk // _PK - t * _HC) * _DP + half
            _acc_row(pk % _PK, lambda k: buf[pl.ds(roff + k * 16, 16)])
            return 0
        lax.fori_loop(0, nj, _edge, 0)
        return 0

    lax.fori_loop(0, _NT, _tile, 0)

    pltpu.sync_copy(amax.at[pl.ds(0, _RW)], mx_hbm.at[pl.ds(lo, _RW)])
    pltpu.sync_copy(amin.at[pl.ds(0, _RW)], mn_hbm.at[pl.ds(lo, _RW)])


def _sc_minmax(hp, src, dst):
    mesh = plsc.VectorSubcoreMesh(core_axis_name="c", subcore_axis_name="s",
                                  num_cores=_NC, num_subcores=_NS)
    run = pl.kernel(
        _sc_minmax_body,
        out_type=(jax.ShapeDtypeStruct((_NPAD, _DP), jnp.int32),
                  jax.ShapeDtypeStruct((_NPAD, _DP), jnp.int32)),
        mesh=mesh,
        scratch_types=[
            pltpu.VMEM((_RW + 1, _DP), jnp.int32),
            pltpu.VMEM((_RW + 1, _DP), jnp.int32),
            pltpu.VMEM((_BUFW,), jnp.int32),
            pltpu.VMEM((_SLTR + 16,), jnp.int32),
            pltpu.VMEM((_DP,), jnp.int32),
            pltpu.SMEM((_NT,), jnp.int32),
            pltpu.SemaphoreType.DMA,
        ],
        compiler_params=pltpu.CompilerParams(needs_layout_passes=False),
    )
    return run(hp, src, dst)


def _unpack(a):
    return jax.lax.bitcast_convert_type(
        a, jnp.bfloat16).reshape(_NPAD, D)[:N]


def _layer(x, src, dst, Wpool, bpool, dww, dwb, Weve, Wself, bias, relu):
    h = _pool_matmul(x, Wpool, bpool)
    hp = jax.lax.bitcast_convert_type(h.reshape(N, _DP, 2),
                                      jnp.int32).reshape(N * _DP)
    hp = jnp.pad(hp, (0, (_NPAD - N) * _DP))
    mxp, mnp = _sc_minmax(hp, src, dst)
    return _out_matmul(x, Wself, _unpack(mxp), _unpack(mnp), Weve, bias,
                       dww, dwb, relu)


def kernel(x, edge_index, c1_Wpool, c1_bpool, c1_dww, c1_dwb, c1_Weve, c1_Wself, c1_bias, c2_Wpool, c2_bpool, c2_dww, c2_dwb, c2_Weve, c2_Wself, c2_bias):
    src = edge_index[0]
    dst = edge_index[1]
    h = _layer(x, src, dst, c1_Wpool, c1_bpool, c1_dww, c1_dwb, c1_Weve,
               c1_Wself, c1_bias, relu=True)
    return _layer(h, src, dst, c2_Wpool, c2_bpool, c2_dww, c2_dwb, c2_Weve,
                  c2_Wself, c2_bias, relu=False)
